# baseline (speedup 1.0000x reference)
import jax
import jax.numpy as jnp
from jax import lax
from jax.experimental import pallas as pl
from jax.experimental.pallas import tpu as pltpu
from jax.experimental.pallas import tpu_sc as plsc
from jax._src.pallas import mpmd as _mpmd

N = 4096
E = 131072
D = 256

NUM_WORKERS = 32
EPW = E // NUM_WORKERS
NBATCH = EPW // 128


def _scores_body(x_ref, w_ref, asrc_ref, adst_ref, s_ref, d_ref, z_ref):
    xb = x_ref[...]
    w = w_ref[...]
    asw = lax.dot_general(asrc_ref[...], w, (((1,), (0,)), ((), ())),
                          preferred_element_type=jnp.float32)
    adw = lax.dot_general(adst_ref[...], w, (((1,), (0,)), ((), ())),
                          preferred_element_type=jnp.float32)
    s_ref[...] = lax.dot_general(asw, xb, (((1,), (1,)), ((), ())),
                                 preferred_element_type=jnp.float32)
    d_ref[...] = lax.dot_general(adw, xb, (((1,), (1,)), ((), ())),
                                 preferred_element_type=jnp.float32)
    z_ref[...] = jnp.zeros_like(z_ref)


def _scores(x, W, a_src, a_dst):
    rb = 512
    zb = N * N // 128 // (N // rb)
    grid = (N // rb,)
    return pl.pallas_call(
        _scores_body,
        grid=grid,
        in_specs=[
            pl.BlockSpec((rb, D), lambda i: (i, 0)),
            pl.BlockSpec((D, D), lambda i: (0, 0)),
            pl.BlockSpec((1, D), lambda i: (0, 0)),
            pl.BlockSpec((1, D), lambda i: (0, 0)),
        ],
        out_specs=[
            pl.BlockSpec((1, rb), lambda i: (0, i)),
            pl.BlockSpec((1, rb), lambda i: (0, i)),
            pl.BlockSpec((zb, 128), lambda i: (i, 0)),
        ],
        out_shape=[
            jax.ShapeDtypeStruct((1, N), jnp.float32),
            jax.ShapeDtypeStruct((1, N), jnp.float32),
            jax.ShapeDtypeStruct((N * N // 128, 128), jnp.float32),
        ],
    )(x, W, a_src, a_dst)


def _project_body(x_ref, w_ref, h_ref):
    h_ref[...] = lax.dot_general(
        x_ref[...], w_ref[...], (((1,), (1,)), ((), ())),
        preferred_element_type=jnp.float32)


def _project(x, W):
    rb = 256
    grid = (N // rb,)
    return pl.pallas_call(
        _project_body,
        grid=grid,
        in_specs=[
            pl.BlockSpec((rb, D), lambda i: (i, 0)),
            pl.BlockSpec((D, D), lambda i: (0, 0)),
        ],
        out_specs=pl.BlockSpec((rb, D), lambda i: (i, 0)),
        out_shape=jax.ShapeDtypeStruct((N, D), jnp.float32),
    )(x, W)


def _scatter_body(a0_ref, s_ref, d_ref, src_ref, dst_ref, out_ref,
                  s_v, d_v, src_v, dst_v, keys_v, vals_v, sem):
    del a0_ref
    c = lax.axis_index("c")
    s_id = lax.axis_index("s")
    wid = c * 16 + s_id
    base = wid * EPW

    pltpu.sync_copy(s_ref.at[0], s_v)
    pltpu.sync_copy(d_ref.at[0], d_v)
    pltpu.sync_copy(src_ref.at[pl.ds(base, EPW)], src_v)
    pltpu.sync_copy(dst_ref.at[pl.ds(base, EPW)], dst_v)

    for b in range(NBATCH):
        for j in range(8):
            off = b * 128 + j * 16
            si = src_v[pl.ds(off, 16)]
            di = dst_v[pl.ds(off, 16)]
            sv = plsc.load_gather(s_v, [si])
            dv = plsc.load_gather(d_v, [di])
            e = sv + dv
            e = jnp.where(e >= 0.0, e, e * jnp.float32(0.2))
            keys_v[pl.ds(off, 16)] = di * N + si
            vals_v[pl.ds(off, 16)] = e
    cp = pltpu.make_async_copy(vals_v, out_ref.at[keys_v], sem)
    cp.start()
    cp.wait()


def _scatter(a0, s, d, src, dst):
    mesh = plsc.VectorSubcoreMesh(core_axis_name="c", subcore_axis_name="s")
    fn = _mpmd._mpmd_map(
        [(mesh, _scatter_body)],
        [jax.ShapeDtypeStruct((N * N,), jnp.float32)],
        input_output_aliases={0: 0},
        scratch_types=[
            pltpu.VMEM((N,), jnp.float32),
            pltpu.VMEM((N,), jnp.float32),
            pltpu.VMEM((EPW,), jnp.int32),
            pltpu.VMEM((EPW,), jnp.int32),
            pltpu.VMEM((EPW,), jnp.int32),
            pltpu.VMEM((EPW,), jnp.float32),
            pltpu.SemaphoreType.DMA,
        ],
        compiler_params=pltpu.CompilerParams(needs_layout_passes=False),
        name="gat_edge_scatter",
    )
    return fn(a0, s, d, src, dst)[0]


RB = 256
G = N // 128


def _attend_body(a_ref, h_ref, hblk_ref, g_ref, b_ref, o_ref):
    A = a_ref[...].reshape(RB, G, 128)
    P = jnp.exp(A) * (A != 0.0).astype(jnp.float32)
    Z = jnp.sum(jnp.sum(P, axis=2), axis=1)
    pos = Z > 0.0
    rZ = jnp.where(pos, jnp.float32(1.0) / Z, jnp.float32(0.0))
    c = jnp.where(pos, jnp.float32(0.0), jnp.float32(1.0 / N))
    alpha = P * rZ[:, None, None] + c[:, None, None]
    alpha = jnp.where(alpha > jnp.float32(1e-6), alpha, jnp.float32(0.0))
    h3 = h_ref[...].reshape(G, 128, D)
    out = jnp.zeros((RB, D), jnp.float32)
    for g in range(G):
        out = out + jnp.dot(alpha[:, g, :], h3[g],
                            preferred_element_type=jnp.float32)
    o = jnp.where(out > 0.0, out, jnp.exp(out) - jnp.float32(1.0))
    y = o + hblk_ref[...]
    mu = jnp.mean(y, axis=1, keepdims=True)
    yc = y - mu
    var = jnp.mean(yc * yc, axis=1, keepdims=True)
    o_ref[...] = (yc / jnp.sqrt(var + jnp.float32(1e-5))) * g_ref[...] + b_ref[...]


def _attend(A2, h, gamma, beta):
    grid = (N // RB,)
    rows = RB * N // 128
    return pl.pallas_call(
        _attend_body,
        grid=grid,
        in_specs=[
            pl.BlockSpec((rows, 128), lambda i: (i, 0)),
            pl.BlockSpec((N, D), lambda i: (0, 0)),
            pl.BlockSpec((RB, D), lambda i: (i, 0)),
            pl.BlockSpec((1, D), lambda i: (0, 0)),
            pl.BlockSpec((1, D), lambda i: (0, 0)),
        ],
        out_specs=pl.BlockSpec((RB, D), lambda i: (i, 0)),
        out_shape=jax.ShapeDtypeStruct((N, D), jnp.float32),
    )(A2, h, h, gamma, beta)


def kernel(x, edge_index, W, a_src, a_dst, gamma, beta):
    src = edge_index[0]
    dst = edge_index[1]
    s, d, a0 = _scores(x, W, a_src, a_dst)
    A = _scatter(a0.reshape(N * N), s, d, src, dst)
    h = _project(x, W)
    A2 = A.reshape(N * N // 128, 128)
    return _attend(A2, h, gamma.reshape(1, D), beta.reshape(1, D))


# speedup vs baseline: 1.0553x; 1.0553x over previous
import jax
import jax.numpy as jnp
from jax import lax
from jax.experimental import pallas as pl
from jax.experimental.pallas import tpu as pltpu
from jax.experimental.pallas import tpu_sc as plsc
from jax._src.pallas import mpmd as _mpmd

N = 4096
E = 131072
D = 256

NUM_WORKERS = 32
EPW = E // NUM_WORKERS
NBATCH = EPW // 128


def _scores_body(x_ref, w_ref, asrc_ref, adst_ref, s_ref, d_ref, z_ref):
    xb = x_ref[...]
    w = w_ref[...]
    asw = lax.dot_general(asrc_ref[...], w, (((1,), (0,)), ((), ())),
                          preferred_element_type=jnp.float32)
    adw = lax.dot_general(adst_ref[...], w, (((1,), (0,)), ((), ())),
                          preferred_element_type=jnp.float32)
    s_ref[...] = lax.dot_general(asw, xb, (((1,), (1,)), ((), ())),
                                 preferred_element_type=jnp.float32)
    d_ref[...] = lax.dot_general(adw, xb, (((1,), (1,)), ((), ())),
                                 preferred_element_type=jnp.float32)
    z_ref[...] = jnp.zeros_like(z_ref)


def _scores(x, W, a_src, a_dst):
    rb = 512
    zb = N * N // 128 // (N // rb)
    grid = (N // rb,)
    return pl.pallas_call(
        _scores_body,
        grid=grid,
        in_specs=[
            pl.BlockSpec((rb, D), lambda i: (i, 0)),
            pl.BlockSpec((D, D), lambda i: (0, 0)),
            pl.BlockSpec((1, D), lambda i: (0, 0)),
            pl.BlockSpec((1, D), lambda i: (0, 0)),
        ],
        out_specs=[
            pl.BlockSpec((1, rb), lambda i: (0, i)),
            pl.BlockSpec((1, rb), lambda i: (0, i)),
            pl.BlockSpec((zb, 128), lambda i: (i, 0)),
        ],
        out_shape=[
            jax.ShapeDtypeStruct((1, N), jnp.float32),
            jax.ShapeDtypeStruct((1, N), jnp.float32),
            jax.ShapeDtypeStruct((N * N // 128, 128), jnp.float32),
        ],
    )(x, W, a_src, a_dst)


def _project_body(x_ref, w_ref, h_ref):
    h_ref[...] = lax.dot_general(
        x_ref[...], w_ref[...], (((1,), (1,)), ((), ())),
        preferred_element_type=jnp.float32)


def _project(x, W):
    rb = 256
    grid = (N // rb,)
    return pl.pallas_call(
        _project_body,
        grid=grid,
        in_specs=[
            pl.BlockSpec((rb, D), lambda i: (i, 0)),
            pl.BlockSpec((D, D), lambda i: (0, 0)),
        ],
        out_specs=pl.BlockSpec((rb, D), lambda i: (i, 0)),
        out_shape=jax.ShapeDtypeStruct((N, D), jnp.float32),
    )(x, W)


def _scatter_body(a0_ref, s_ref, d_ref, src_ref, dst_ref, out_ref,
                  s_v, d_v, src_v, dst_v, keys_v, vals_v, sem):
    del a0_ref
    c = lax.axis_index("c")
    s_id = lax.axis_index("s")
    wid = c * 16 + s_id
    base = wid * EPW

    pltpu.sync_copy(s_ref.at[0], s_v)
    pltpu.sync_copy(d_ref.at[0], d_v)
    pltpu.sync_copy(src_ref.at[pl.ds(base, EPW)], src_v)
    pltpu.sync_copy(dst_ref.at[pl.ds(base, EPW)], dst_v)

    for b in range(NBATCH):
        for j in range(8):
            off = b * 128 + j * 16
            si = src_v[pl.ds(off, 16)]
            di = dst_v[pl.ds(off, 16)]
            sv = plsc.load_gather(s_v, [si])
            dv = plsc.load_gather(d_v, [di])
            e = sv + dv
            e = jnp.where(e >= 0.0, e, e * jnp.float32(0.2))
            keys_v[pl.ds(off, 16)] = di * N + si
            vals_v[pl.ds(off, 16)] = e
    cp = pltpu.make_async_copy(vals_v, out_ref.at[keys_v], sem)
    cp.start()
    cp.wait()


def _scatter(a0, s, d, src, dst):
    mesh = plsc.VectorSubcoreMesh(core_axis_name="c", subcore_axis_name="s")
    fn = _mpmd._mpmd_map(
        [(mesh, _scatter_body)],
        [jax.ShapeDtypeStruct((N * N,), jnp.float32)],
        input_output_aliases={0: 0},
        scratch_types=[
            pltpu.VMEM((N,), jnp.float32),
            pltpu.VMEM((N,), jnp.float32),
            pltpu.VMEM((EPW,), jnp.int32),
            pltpu.VMEM((EPW,), jnp.int32),
            pltpu.VMEM((EPW,), jnp.int32),
            pltpu.VMEM((EPW,), jnp.float32),
            pltpu.SemaphoreType.DMA,
        ],
        compiler_params=pltpu.CompilerParams(needs_layout_passes=False),
        name="gat_edge_scatter",
    )
    return fn(a0, s, d, src, dst)[0]


RB = 512
G = N // 128


def _attend_body(a_ref, h_ref, hblk_ref, g_ref, b_ref, o_ref):
    A = a_ref[...].reshape(RB, G, 128)
    L = jnp.where(A == 0.0, jnp.float32(-1e9), A)
    m = jnp.max(jnp.max(L, axis=2), axis=1)
    P = jnp.exp(L - m[:, None, None])
    Z = jnp.sum(jnp.sum(P, axis=2), axis=1)
    rZ = (jnp.float32(1.0) / Z)[:, None, None]
    alpha = P * rZ
    alpha = jnp.where(alpha > jnp.float32(1e-6), alpha, jnp.float32(0.0))
    h3 = h_ref[...].reshape(G, 128, D)
    out = jnp.zeros((RB, D), jnp.float32)
    for g in range(G):
        out = out + jnp.dot(alpha[:, g, :], h3[g],
                            preferred_element_type=jnp.float32)
    o = jnp.where(out > 0.0, out, jnp.exp(out) - jnp.float32(1.0))
    y = o + hblk_ref[...]
    mu = jnp.mean(y, axis=1, keepdims=True)
    yc = y - mu
    var = jnp.mean(yc * yc, axis=1, keepdims=True)
    o_ref[...] = (yc / jnp.sqrt(var + jnp.float32(1e-5))) * g_ref[...] + b_ref[...]


def _attend(A2, h, gamma, beta):
    grid = (N // RB,)
    rows = RB * N // 128
    return pl.pallas_call(
        _attend_body,
        grid=grid,
        in_specs=[
            pl.BlockSpec((rows, 128), lambda i: (i, 0)),
            pl.BlockSpec((N, D), lambda i: (0, 0)),
            pl.BlockSpec((RB, D), lambda i: (i, 0)),
            pl.BlockSpec((1, D), lambda i: (0, 0)),
            pl.BlockSpec((1, D), lambda i: (0, 0)),
        ],
        out_specs=pl.BlockSpec((RB, D), lambda i: (i, 0)),
        out_shape=jax.ShapeDtypeStruct((N, D), jnp.float32),
    )(A2, h, h, gamma, beta)


def kernel(x, edge_index, W, a_src, a_dst, gamma, beta):
    src = edge_index[0]
    dst = edge_index[1]
    s, d, a0 = _scores(x, W, a_src, a_dst)
    A = _scatter(a0.reshape(N * N), s, d, src, dst)
    h = _project(x, W)
    A2 = A.reshape(N * N // 128, 128)
    return _attend(A2, h, gamma.reshape(1, D), beta.reshape(1, D))
